# baseline (speedup 1.0000x reference)
import functools

import jax
import jax.numpy as jnp
from jax.experimental import pallas as pl
from jax.experimental.pallas import tpu as pltpu

_K = 64
_VT = 4096
_BIG = 3.0e38


def _stage(a, j, s, invert=False):
    n = a.shape[0]
    rest = a.shape[1:]
    if n >= 2 * s:
        ar = a.reshape((n // (2 * s), 2, s // (2 * j), 2, j) + rest)
        A = ar[:, :, :, 0]
        B = ar[:, :, :, 1]
        mn = jnp.minimum(A, B)
        mx = jnp.maximum(A, B)
        lo, hi = (mx, mn) if invert else (mn, mx)
        newA = jnp.stack([lo[:, 0], hi[:, 1]], axis=1)
        newB = jnp.stack([hi[:, 0], lo[:, 1]], axis=1)
        return jnp.stack([newA, newB], axis=3).reshape((n,) + rest)
    ar = a.reshape((s // (2 * j), 2, j) + rest)
    A = ar[:, 0]
    B = ar[:, 1]
    mn = jnp.minimum(A, B)
    mx = jnp.maximum(A, B)
    lo, hi = (mx, mn) if invert else (mn, mx)
    return jnp.stack([lo, hi], axis=1).reshape((n,) + rest)


def _sort_full(a, invert=False):
    n = a.shape[0]
    s = 2
    while s <= n:
        j = s >> 1
        while j >= 1:
            a = _stage(a, j, s, invert)
            j >>= 1
        s <<= 1
    return a


def _cleanup(a, run, invert=False):
    j = run >> 1
    while j >= 1:
        a = _stage(a, j, run, invert)
        j >>= 1
    return a


def _topk_body(v_real, x_ref, b_ref, out_ref, s_ref):
    i = pl.program_id(0)
    nt = pl.num_programs(0)
    Q = x_ref.shape[0]

    @pl.when(i == 0)
    def _init():
        s_ref[...] = jnp.full((_K, Q), _BIG, jnp.float32)

    x = x_ref[...]
    bt = b_ref[...]
    sq_x = jnp.sum(x * x, axis=1)
    sq_b = jnp.sum(bt * bt, axis=1)
    prod = jax.lax.dot_general(bt, x, (((1,), (1,)), ((), ())),
                               preferred_element_type=jnp.float32)
    d = sq_b[:, None] + sq_x[None, :] - 2.0 * prod

    row = jax.lax.broadcasted_iota(jnp.int32, (_VT, 1), 0) + i * _VT
    d = jnp.where(row >= v_real, _BIG, d)

    r = _VT // _K
    u = d.reshape(_K, r, Q)
    lo = _sort_full(u[:, : r // 2])
    hi = _sort_full(u[:, r // 2:], invert=True)
    m = jnp.minimum(lo, hi)
    r //= 2
    while r > 1:
        lo = _cleanup(m[:, : r // 2], _K)
        hi = _cleanup(m[:, r // 2:], _K, invert=True)
        m = jnp.minimum(lo, hi)
        r //= 2
    tile_top = _cleanup(m, _K, invert=True)[:, 0]

    merged = jnp.minimum(s_ref[...], tile_top)
    s_ref[...] = _cleanup(merged, _K)

    @pl.when(i == nt - 1)
    def _done():
        out_ref[...] = s_ref[...]


def kernel(input, target, b):
    n, dim = input.shape
    v = b.shape[0]
    nt = (v + _VT - 1) // _VT
    v_pad = nt * _VT
    b_p = jnp.pad(b, ((0, v_pad - v), (0, 0)))
    out = pl.pallas_call(
        functools.partial(_topk_body, v),
        grid=(nt,),
        in_specs=[
            pl.BlockSpec((n, dim), lambda i: (0, 0)),
            pl.BlockSpec((_VT, dim), lambda i: (i, 0)),
        ],
        out_specs=pl.BlockSpec((_K, n), lambda i: (0, 0)),
        out_shape=jax.ShapeDtypeStruct((_K, n), jnp.float32),
        scratch_shapes=[pltpu.VMEM((_K, n), jnp.float32)],
        compiler_params=pltpu.CompilerParams(
            dimension_semantics=("arbitrary",),
        ),
    )(input, b_p)
    return out.T


# speedup vs baseline: 1.0282x; 1.0282x over previous
import functools

import jax
import jax.numpy as jnp
from jax.experimental import pallas as pl
from jax.experimental.pallas import tpu as pltpu

_K = 64
_VT = 2048
_BIG = 3.0e38


def _stage(a, j, s, invert=False):
    n, Q = a.shape
    if n >= 2 * s:
        ar = a.reshape(n // (2 * s), 2, s // (2 * j), 2, j, Q)
        A = ar[:, :, :, 0]
        B = ar[:, :, :, 1]
        mn = jnp.minimum(A, B)
        mx = jnp.maximum(A, B)
        lo, hi = (mx, mn) if invert else (mn, mx)
        newA = jnp.stack([lo[:, 0], hi[:, 1]], axis=1)
        newB = jnp.stack([hi[:, 0], lo[:, 1]], axis=1)
        return jnp.stack([newA, newB], axis=3).reshape(n, Q)
    ar = a.reshape(s // (2 * j), 2, j, Q)
    A = ar[:, 0]
    B = ar[:, 1]
    mn = jnp.minimum(A, B)
    mx = jnp.maximum(A, B)
    lo, hi = (mx, mn) if invert else (mn, mx)
    return jnp.stack([lo, hi], axis=1).reshape(n, Q)


def _sort_runs(a, run):
    s = 2
    while s <= run:
        j = s >> 1
        while j >= 1:
            a = _stage(a, j, s)
            j >>= 1
        s <<= 1
    return a


def _cleanup(a, run, invert=False):
    j = run >> 1
    while j >= 1:
        a = _stage(a, j, run, invert)
        j >>= 1
    return a


def _pair_min(a):
    n, Q = a.shape
    ar = a.reshape(n // (2 * _K), 2, _K, Q)
    return jnp.minimum(ar[:, 0], ar[:, 1]).reshape(n // 2, Q)


def _topk_body(v_real, half, x_ref, b_ref, out_ref, s_ref):
    c = pl.program_id(0)
    i = pl.program_id(1)
    nt = pl.num_programs(1)
    Q = x_ref.shape[0]

    @pl.when(i == 0)
    def _init():
        s_ref[...] = jnp.full((_K, Q), _BIG, jnp.float32)

    x = x_ref[...]
    bt = b_ref[0]
    sq_x = jnp.sum(x * x, axis=1)
    sq_b = jnp.sum(bt * bt, axis=1)
    prod = jax.lax.dot_general(bt, x, (((1,), (1,)), ((), ())),
                               preferred_element_type=jnp.float32)
    d = sq_b[:, None] + sq_x[None, :] - 2.0 * prod

    row = (jax.lax.broadcasted_iota(jnp.int32, (_VT, 1), 0)
           + c * half + i * _VT)
    d = jnp.where(row >= v_real, _BIG, d)

    d = _sort_runs(d, _K)
    n = _VT
    while n > _K:
        d = _pair_min(d)
        n //= 2
        d = _cleanup(d, _K, invert=(n == _K))

    merged = jnp.minimum(s_ref[...], d)
    s_ref[...] = _cleanup(merged, _K)

    @pl.when(i == nt - 1)
    def _done():
        out_ref[0] = s_ref[...]


def _merge_body(h_ref, out_ref):
    a = h_ref[0]
    bd = _cleanup(h_ref[1], _K, invert=True)
    out_ref[...] = _cleanup(jnp.minimum(a, bd), _K)


def kernel(input, target, b):
    n, dim = input.shape
    v = b.shape[0]
    nt = (v // 2 + _VT - 1) // _VT
    half = nt * _VT
    b_p = jnp.pad(b, ((0, 2 * half - v), (0, 0))).reshape(2, half, dim)
    halves = pl.pallas_call(
        functools.partial(_topk_body, v, half),
        grid=(2, nt),
        in_specs=[
            pl.BlockSpec((n, dim), lambda c, i: (0, 0)),
            pl.BlockSpec((1, _VT, dim), lambda c, i: (c, i, 0)),
        ],
        out_specs=pl.BlockSpec((1, _K, n), lambda c, i: (c, 0, 0)),
        out_shape=jax.ShapeDtypeStruct((2, _K, n), jnp.float32),
        scratch_shapes=[pltpu.VMEM((_K, n), jnp.float32)],
        compiler_params=pltpu.CompilerParams(
            dimension_semantics=("parallel", "arbitrary"),
        ),
    )(input, b_p)
    out = pl.pallas_call(
        _merge_body,
        out_shape=jax.ShapeDtypeStruct((_K, n), jnp.float32),
    )(halves)
    return out.T


# speedup vs baseline: 1.1285x; 1.0976x over previous
import functools

import jax
import jax.numpy as jnp
from jax.experimental import pallas as pl
from jax.experimental.pallas import tpu as pltpu

_K = 64
_VT = 4096
_BIG = 3.0e38


def _bitonic_list(vals, invert=False):
    n = len(vals)
    s = 2
    while s <= n:
        j = s >> 1
        while j >= 1:
            new = list(vals)
            for i in range(n):
                p = i ^ j
                if p > i:
                    up = ((i & s) == 0) != invert
                    mn = jnp.minimum(vals[i], vals[p])
                    mx = jnp.maximum(vals[i], vals[p])
                    new[i], new[p] = (mn, mx) if up else (mx, mn)
            vals = new
            j >>= 1
        s <<= 1
    return vals


def _cleanup_list(vals, invert=False):
    n = len(vals)
    j = n >> 1
    while j >= 1:
        new = list(vals)
        for i in range(n):
            p = i ^ j
            if p > i:
                mn = jnp.minimum(vals[i], vals[p])
                mx = jnp.maximum(vals[i], vals[p])
                new[i], new[p] = (mx, mn) if invert else (mn, mx)
        vals = new
        j >>= 1
    return vals


def _stage(a, j, s, invert=False):
    n = a.shape[0]
    rest = a.shape[1:]
    if n >= 2 * s:
        ar = a.reshape((n // (2 * s), 2, s // (2 * j), 2, j) + rest)
        A = ar[:, :, :, 0]
        B = ar[:, :, :, 1]
        mn = jnp.minimum(A, B)
        mx = jnp.maximum(A, B)
        lo, hi = (mx, mn) if invert else (mn, mx)
        newA = jnp.stack([lo[:, 0], hi[:, 1]], axis=1)
        newB = jnp.stack([hi[:, 0], lo[:, 1]], axis=1)
        return jnp.stack([newA, newB], axis=3).reshape((n,) + rest)
    ar = a.reshape((s // (2 * j), 2, j) + rest)
    A = ar[:, 0]
    B = ar[:, 1]
    mn = jnp.minimum(A, B)
    mx = jnp.maximum(A, B)
    lo, hi = (mx, mn) if invert else (mn, mx)
    return jnp.stack([lo, hi], axis=1).reshape((n,) + rest)


def _cleanup(a, run, invert=False):
    j = run >> 1
    while j >= 1:
        a = _stage(a, j, run, invert)
        j >>= 1
    return a


def _topk_body(v_real, x_ref, b_ref, out_ref, s_ref):
    i = pl.program_id(0)
    nt = pl.num_programs(0)
    Q = x_ref.shape[0]

    @pl.when(i == 0)
    def _init():
        s_ref[...] = jnp.full((_K, Q), _BIG, jnp.float32)

    x = x_ref[...]
    bt = b_ref[...]
    sq_x = jnp.sum(x * x, axis=1)
    sq_b = jnp.sum(bt * bt, axis=1)
    prod = jax.lax.dot_general(bt, x, (((1,), (1,)), ((), ())),
                               preferred_element_type=jnp.float32)
    d = sq_b[:, None] + sq_x[None, :] - 2.0 * prod

    row = jax.lax.broadcasted_iota(jnp.int32, (_VT, 1), 0) + i * _VT
    d = jnp.where(row >= v_real, _BIG, d)

    nb = _VT // _K
    lo = [d[u * _K: u * _K + _K // 2] for u in range(nb)]
    hi = [d[u * _K + _K // 2: (u + 1) * _K] for u in range(nb)]
    lo = _bitonic_list(lo)
    hi = _bitonic_list(hi, invert=True)
    m = [jnp.minimum(a, b) for a, b in zip(lo, hi)]
    w = _K // 2
    while w > 8:
        half = w // 2
        mlo = _cleanup_list([v[:half] for v in m])
        mhi = _cleanup_list([v[half:] for v in m], invert=True)
        m = [jnp.minimum(a, b) for a, b in zip(mlo, mhi)]
        w = half
    arr = jnp.stack(m, axis=0)
    while w > 1:
        half = w // 2
        lo3 = _cleanup(arr[:, :half], _K)
        hi3 = _cleanup(arr[:, half:], _K, invert=True)
        arr = jnp.minimum(lo3, hi3)
        w = half
    tile_top = _cleanup(arr, _K, invert=True)[:, 0]

    merged = jnp.minimum(s_ref[...], tile_top)
    s_ref[...] = _cleanup(merged, _K)

    @pl.when(i == nt - 1)
    def _done():
        out_ref[...] = s_ref[...]


def kernel(input, target, b):
    n, dim = input.shape
    v = b.shape[0]
    nt = (v + _VT - 1) // _VT
    v_pad = nt * _VT
    b_p = jnp.pad(b, ((0, v_pad - v), (0, 0)))
    out = pl.pallas_call(
        functools.partial(_topk_body, v),
        grid=(nt,),
        in_specs=[
            pl.BlockSpec((n, dim), lambda i: (0, 0)),
            pl.BlockSpec((_VT, dim), lambda i: (i, 0)),
        ],
        out_specs=pl.BlockSpec((_K, n), lambda i: (0, 0)),
        out_shape=jax.ShapeDtypeStruct((_K, n), jnp.float32),
        scratch_shapes=[pltpu.VMEM((_K, n), jnp.float32)],
        compiler_params=pltpu.CompilerParams(
            dimension_semantics=("arbitrary",),
        ),
    )(input, b_p)
    return out.T


# speedup vs baseline: 1.2391x; 1.0980x over previous
import functools

import jax
import jax.numpy as jnp
from jax.experimental import pallas as pl
from jax.experimental.pallas import tpu as pltpu

_K = 64
_VT = 4096
_BIG = 3.0e38


def _oddeven_sort_list(vals, invert=False):
    vals = list(vals)
    n = len(vals)
    p = 1
    while p < n:
        k = p
        while k >= 1:
            for j in range(k % p, n - k, 2 * k):
                for i in range(min(k, n - j - k)):
                    a, b = i + j, i + j + k
                    if a // (2 * p) == b // (2 * p):
                        mn = jnp.minimum(vals[a], vals[b])
                        mx = jnp.maximum(vals[a], vals[b])
                        vals[a], vals[b] = (mx, mn) if invert else (mn, mx)
            k >>= 1
        p <<= 1
    return vals


def _cleanup_list(vals, invert=False):
    n = len(vals)
    j = n >> 1
    while j >= 1:
        new = list(vals)
        for i in range(n):
            p = i ^ j
            if p > i:
                mn = jnp.minimum(vals[i], vals[p])
                mx = jnp.maximum(vals[i], vals[p])
                new[i], new[p] = (mx, mn) if invert else (mn, mx)
        vals = new
        j >>= 1
    return vals


def _stage(a, j, s, invert=False):
    n = a.shape[0]
    rest = a.shape[1:]
    if n >= 2 * s:
        ar = a.reshape((n // (2 * s), 2, s // (2 * j), 2, j) + rest)
        A = ar[:, :, :, 0]
        B = ar[:, :, :, 1]
        mn = jnp.minimum(A, B)
        mx = jnp.maximum(A, B)
        lo, hi = (mx, mn) if invert else (mn, mx)
        newA = jnp.stack([lo[:, 0], hi[:, 1]], axis=1)
        newB = jnp.stack([hi[:, 0], lo[:, 1]], axis=1)
        return jnp.stack([newA, newB], axis=3).reshape((n,) + rest)
    ar = a.reshape((s // (2 * j), 2, j) + rest)
    A = ar[:, 0]
    B = ar[:, 1]
    mn = jnp.minimum(A, B)
    mx = jnp.maximum(A, B)
    lo, hi = (mx, mn) if invert else (mn, mx)
    return jnp.stack([lo, hi], axis=1).reshape((n,) + rest)


def _cleanup(a, run, invert=False):
    j = run >> 1
    while j >= 1:
        a = _stage(a, j, run, invert)
        j >>= 1
    return a


def _topk_body(v_real, x_ref, b_ref, out_ref, s_ref):
    i = pl.program_id(0)
    nt = pl.num_programs(0)
    Q = x_ref.shape[0]

    @pl.when(i == 0)
    def _init():
        s_ref[...] = jnp.full((_K, Q), _BIG, jnp.float32)

    x = x_ref[...]
    bt = b_ref[...]
    sq_x = jnp.sum(x * x, axis=1)
    sq_b = jnp.sum(bt * bt, axis=1)
    prod = jax.lax.dot_general(bt, x, (((1,), (1,)), ((), ())),
                               preferred_element_type=jnp.float32)
    d = sq_b[:, None] + sq_x[None, :] - 2.0 * prod

    row = jax.lax.broadcasted_iota(jnp.int32, (_VT, 1), 0) + i * _VT
    d = jnp.where(row >= v_real, _BIG, d)

    nb = _VT // _K
    lo = [d[u * _K: u * _K + _K // 2] for u in range(nb)]
    hi = [d[u * _K + _K // 2: (u + 1) * _K] for u in range(nb)]
    lo = _oddeven_sort_list(lo)
    hi = _oddeven_sort_list(hi, invert=True)
    m = [jnp.minimum(a, b) for a, b in zip(lo, hi)]
    w = _K // 2
    while w > 8:
        half = w // 2
        mlo = _cleanup_list([v[:half] for v in m])
        mhi = _cleanup_list([v[half:] for v in m], invert=True)
        m = [jnp.minimum(a, b) for a, b in zip(mlo, mhi)]
        w = half
    arr = jnp.stack(m, axis=0)
    while w > 1:
        half = w // 2
        lo3 = _cleanup(arr[:, :half], _K)
        hi3 = _cleanup(arr[:, half:], _K, invert=True)
        arr = jnp.minimum(lo3, hi3)
        w = half
    tile_top = _cleanup(arr, _K, invert=True)[:, 0]

    merged = jnp.minimum(s_ref[...], tile_top)
    s_ref[...] = _cleanup(merged, _K)

    @pl.when(i == nt - 1)
    def _done():
        out_ref[...] = s_ref[...]


def kernel(input, target, b):
    n, dim = input.shape
    v = b.shape[0]
    nt = (v + _VT - 1) // _VT
    v_pad = nt * _VT
    b_p = jnp.pad(b, ((0, v_pad - v), (0, 0)))
    out = pl.pallas_call(
        functools.partial(_topk_body, v),
        grid=(nt,),
        in_specs=[
            pl.BlockSpec((n, dim), lambda i: (0, 0)),
            pl.BlockSpec((_VT, dim), lambda i: (i, 0)),
        ],
        out_specs=pl.BlockSpec((_K, n), lambda i: (0, 0)),
        out_shape=jax.ShapeDtypeStruct((_K, n), jnp.float32),
        scratch_shapes=[pltpu.VMEM((_K, n), jnp.float32)],
        compiler_params=pltpu.CompilerParams(
            dimension_semantics=("arbitrary",),
        ),
    )(input, b_p)
    return out.T


# speedup vs baseline: 1.3014x; 1.0502x over previous
import functools

import jax
import jax.numpy as jnp
from jax.experimental import pallas as pl
from jax.experimental.pallas import tpu as pltpu

_K = 64
_VT = 4096
_BIG = 3.0e38


def _oddeven_sort_list(vals, invert=False):
    vals = list(vals)
    n = len(vals)
    p = 1
    while p < n:
        k = p
        while k >= 1:
            for j in range(k % p, n - k, 2 * k):
                for i in range(min(k, n - j - k)):
                    a, b = i + j, i + j + k
                    if a // (2 * p) == b // (2 * p):
                        mn = jnp.minimum(vals[a], vals[b])
                        mx = jnp.maximum(vals[a], vals[b])
                        vals[a], vals[b] = (mx, mn) if invert else (mn, mx)
            k >>= 1
        p <<= 1
    return vals


def _cleanup_list(vals, invert=False):
    n = len(vals)
    j = n >> 1
    while j >= 1:
        new = list(vals)
        for i in range(n):
            p = i ^ j
            if p > i:
                mn = jnp.minimum(vals[i], vals[p])
                mx = jnp.maximum(vals[i], vals[p])
                new[i], new[p] = (mx, mn) if invert else (mn, mx)
        vals = new
        j >>= 1
    return vals


def _stage(a, j, s, invert=False):
    n = a.shape[0]
    rest = a.shape[1:]
    if n >= 2 * s:
        ar = a.reshape((n // (2 * s), 2, s // (2 * j), 2, j) + rest)
        A = ar[:, :, :, 0]
        B = ar[:, :, :, 1]
        mn = jnp.minimum(A, B)
        mx = jnp.maximum(A, B)
        lo, hi = (mx, mn) if invert else (mn, mx)
        newA = jnp.stack([lo[:, 0], hi[:, 1]], axis=1)
        newB = jnp.stack([hi[:, 0], lo[:, 1]], axis=1)
        return jnp.stack([newA, newB], axis=3).reshape((n,) + rest)
    ar = a.reshape((s // (2 * j), 2, j) + rest)
    A = ar[:, 0]
    B = ar[:, 1]
    mn = jnp.minimum(A, B)
    mx = jnp.maximum(A, B)
    lo, hi = (mx, mn) if invert else (mn, mx)
    return jnp.stack([lo, hi], axis=1).reshape((n,) + rest)


def _cleanup(a, run, invert=False):
    j = run >> 1
    while j >= 1:
        a = _stage(a, j, run, invert)
        j >>= 1
    return a


def _topk_body(v_real, x_ref, b_ref, out_ref, s_ref):
    i = pl.program_id(0)
    nt = pl.num_programs(0)
    Q = x_ref.shape[0]

    @pl.when(i == 0)
    def _init():
        s_ref[...] = jnp.full((_K, Q), _BIG, jnp.float32)

    x = x_ref[...]
    bt = b_ref[...]
    sq_x = jnp.sum(x * x, axis=1)
    sq_b = jnp.sum(bt * bt, axis=1)

    row = jax.lax.broadcasted_iota(jnp.int32, (_VT, 1), 0)[:, 0] + i * _VT
    sq_b = jnp.where(row >= v_real, _BIG, sq_b)

    prod = jax.lax.dot_general(bt, x, (((1,), (1,)), ((), ())),
                               preferred_element_type=jnp.float32)
    d = sq_b[:, None] + sq_x[None, :] - 2.0 * prod

    nb = _VT // _K
    lo = [d[u * _K: u * _K + _K // 2] for u in range(nb)]
    hi = [d[u * _K + _K // 2: (u + 1) * _K] for u in range(nb)]
    lo = _oddeven_sort_list(lo)
    hi = _oddeven_sort_list(hi, invert=True)
    m = [jnp.minimum(a, b) for a, b in zip(lo, hi)]
    w = _K // 2
    while w > 8:
        half = w // 2
        mlo = _cleanup_list([v[:half] for v in m])
        mhi = _cleanup_list([v[half:] for v in m], invert=True)
        m = [jnp.minimum(a, b) for a, b in zip(mlo, mhi)]
        w = half
    arr = jnp.stack(m, axis=0)
    while w > 1:
        half = w // 2
        lo3 = _cleanup(arr[:, :half], _K)
        hi3 = _cleanup(arr[:, half:], _K, invert=True)
        arr = jnp.minimum(lo3, hi3)
        w = half
    tile_top = _cleanup(arr, _K, invert=True)[:, 0]

    merged = jnp.minimum(s_ref[...], tile_top)
    s_ref[...] = _cleanup(merged, _K)

    @pl.when(i == nt - 1)
    def _done():
        out_ref[...] = s_ref[...]


def kernel(input, target, b):
    n, dim = input.shape
    v = b.shape[0]
    nt = (v + _VT - 1) // _VT
    v_pad = nt * _VT
    b_p = jnp.pad(b, ((0, v_pad - v), (0, 0)))
    out = pl.pallas_call(
        functools.partial(_topk_body, v),
        grid=(nt,),
        in_specs=[
            pl.BlockSpec((n, dim), lambda i: (0, 0)),
            pl.BlockSpec((_VT, dim), lambda i: (i, 0)),
        ],
        out_specs=pl.BlockSpec((_K, n), lambda i: (0, 0)),
        out_shape=jax.ShapeDtypeStruct((_K, n), jnp.float32),
        scratch_shapes=[pltpu.VMEM((_K, n), jnp.float32)],
        compiler_params=pltpu.CompilerParams(
            dimension_semantics=("arbitrary",),
        ),
    )(input, b_p)
    return out.T
